# baseline (speedup 1.0000x reference)
import functools

import jax
import jax.numpy as jnp
from jax import lax
from jax.experimental import pallas as pl
from jax.experimental.pallas import tpu as pltpu
from jax.experimental.pallas import tpu_sc as plsc

N_NODES = 10000
N_EDGES = 320000
F = 128

NC = 2
NS = 16
NW = NC * NS
CH = 128
CPB = 8
NBLK = 10
EPWP = NBLK * CPB * CH
E_PAD = NW * EPWP
N_DUMMY = 512
N_ACC = N_NODES + N_DUMMY
CPS = 624
TAIL = N_NODES - CPS * NS
TAIL_OFF = CPS * NS

_mesh = plsc.VectorSubcoreMesh(core_axis_name="c", subcore_axis_name="s")


@functools.partial(
    pl.kernel,
    out_type=jax.ShapeDtypeStruct((NC, N_NODES, F), jnp.float32),
    mesh=_mesh,
    scratch_types=[
        pltpu.VMEM_SHARED((N_ACC, F), jnp.float32),
        pltpu.VMEM((CPB, CH), jnp.int32),
        pltpu.VMEM((CPB, CH), jnp.int32),
        pltpu.VMEM((CPB, CH), jnp.int32),
        pltpu.VMEM((CPB, CH), jnp.int32),
        pltpu.VMEM((CH, F), jnp.float32),
        pltpu.VMEM((CH, F), jnp.float32),
        pltpu.SemaphoreType.DMA,
        pltpu.SemaphoreType.DMA,
        pltpu.SemaphoreType.DMA,
        pltpu.SemaphoreType.DMA,
    ],
)
def _seg_sum(table, zeros, src4, dst4, out, acc, sA, dA, sB, dB, rows0, rows1,
             semiA, semiB, semg0, semg1):
    c = lax.axis_index("c")
    s = lax.axis_index("s")
    w = s * NC + c

    off = pl.multiple_of(s * CPS, 8)
    pltpu.sync_copy(zeros.at[pl.ds(off, CPS)], acc.at[pl.ds(off, CPS)])

    @pl.when(s == NS - 1)
    def _():
        pltpu.sync_copy(zeros.at[pl.ds(TAIL_OFF, TAIL)],
                        acc.at[pl.ds(TAIL_OFF, TAIL)])

    plsc.subcore_barrier()

    rows = (rows0, rows1)
    semg = (semg0, semg1)

    pltpu.async_copy(src4.at[w, 0], sA, semiA)
    pltpu.async_copy(dst4.at[w, 0], dA, semiA)
    pltpu.make_async_copy(src4.at[w, 0], sA, semiA).wait()
    pltpu.make_async_copy(dst4.at[w, 0], dA, semiA).wait()
    pltpu.async_copy(table.at[sA.at[0]], rows0, semg0)

    def pair(t, carry):
        be = 2 * t

        pltpu.async_copy(src4.at[w, be + 1], sB, semiB)
        pltpu.async_copy(dst4.at[w, be + 1], dB, semiB)
        for r in range(CPB):
            cur, nxt = rows[r % 2], rows[(r + 1) % 2]
            if r < CPB - 1:
                pltpu.async_copy(table.at[sA.at[r + 1]], nxt, semg[(r + 1) % 2])
            else:
                pltpu.make_async_copy(src4.at[w, be + 1], sB, semiB).wait()
                pltpu.make_async_copy(dst4.at[w, be + 1], dB, semiB).wait()
                pltpu.async_copy(table.at[sB.at[0]], nxt, semg[(r + 1) % 2])
            pltpu.make_async_copy(table.at[sA.at[r]], cur, semg[r % 2]).wait()
            pltpu.sync_copy(cur, acc.at[dA.at[r]], add=True)

        @pl.when(t < NBLK // 2 - 1)
        def _():
            pltpu.async_copy(src4.at[w, be + 2], sA, semiA)
            pltpu.async_copy(dst4.at[w, be + 2], dA, semiA)

        for r in range(CPB):
            cur, nxt = rows[r % 2], rows[(r + 1) % 2]
            if r < CPB - 1:
                pltpu.async_copy(table.at[sB.at[r + 1]], nxt, semg[(r + 1) % 2])
            else:
                @pl.when(t < NBLK // 2 - 1)
                def _():
                    pltpu.make_async_copy(src4.at[w, be + 2], sA, semiA).wait()
                    pltpu.make_async_copy(dst4.at[w, be + 2], dA, semiA).wait()
                    pltpu.async_copy(table.at[sA.at[0]], nxt,
                                     semg[(r + 1) % 2])
            pltpu.make_async_copy(table.at[sB.at[r]], cur, semg[r % 2]).wait()
            pltpu.sync_copy(cur, acc.at[dB.at[r]], add=True)
        return carry

    lax.fori_loop(0, NBLK // 2, pair, 0)
    plsc.subcore_barrier()

    pltpu.sync_copy(acc.at[pl.ds(off, CPS)], out.at[c, pl.ds(off, CPS)])

    @pl.when(s == NS - 1)
    def _():
        pltpu.sync_copy(acc.at[pl.ds(TAIL_OFF, TAIL)],
                        out.at[c, pl.ds(TAIL_OFF, TAIL)])


R = 1000


def _lin_relu_body(x_ref, agg_ref, w_ref, b_ref, o_ref):
    a = x_ref[...] + agg_ref[0] + agg_ref[1]
    h = jnp.dot(a, w_ref[...], preferred_element_type=jnp.float32) + b_ref[...]
    o_ref[...] = jnp.maximum(h, 0.0)


def _lin_relu(x, agg, W, b):
    return pl.pallas_call(
        _lin_relu_body,
        grid=(N_NODES // R,),
        in_specs=[
            pl.BlockSpec((R, F), lambda i: (i, 0)),
            pl.BlockSpec((NC, R, F), lambda i: (0, i, 0)),
            pl.BlockSpec((F, F), lambda i: (0, 0)),
            pl.BlockSpec((1, F), lambda i: (0, 0)),
        ],
        out_specs=pl.BlockSpec((R, F), lambda i: (i, 0)),
        out_shape=jax.ShapeDtypeStruct((N_NODES, F), jnp.float32),
    )(x, agg, W, b)


def _final_body(h_ref, agg_ref, w2_ref, b2_ref, wf1_ref, bf1_ref, wf2_ref,
                bf2_ref, o_ref, acc_ref):
    i = pl.program_id(0)
    a = h_ref[...] + agg_ref[0] + agg_ref[1]
    h2 = jnp.dot(a, w2_ref[...], preferred_element_type=jnp.float32) + b2_ref[...]
    h2 = jnp.maximum(h2, 0.0)
    part = jnp.sum(h2, axis=0, keepdims=True)

    @pl.when(i == 0)
    def _():
        acc_ref[0:1] = part

    @pl.when(i > 0)
    def _():
        acc_ref[0:1] = acc_ref[0:1] + part

    @pl.when(i == pl.num_programs(0) - 1)
    def _():
        hg = jnp.dot(acc_ref[0:1], wf1_ref[...],
                     preferred_element_type=jnp.float32) + bf1_ref[...]
        hg = jnp.maximum(hg, 0.0)
        z = jnp.sum(hg * wf2_ref[...], axis=1, keepdims=True) + bf2_ref[...]
        o_ref[...] = 1.0 / (1.0 + jnp.exp(-z))


def _final(h, agg, W2, b2, Wf1, bf1, Wf2, bf2):
    return pl.pallas_call(
        _final_body,
        grid=(N_NODES // R,),
        in_specs=[
            pl.BlockSpec((R, F), lambda i: (i, 0)),
            pl.BlockSpec((NC, R, F), lambda i: (0, i, 0)),
            pl.BlockSpec((F, F), lambda i: (0, 0)),
            pl.BlockSpec((1, F), lambda i: (0, 0)),
            pl.BlockSpec((F, F), lambda i: (0, 0)),
            pl.BlockSpec((1, F), lambda i: (0, 0)),
            pl.BlockSpec((1, F), lambda i: (0, 0)),
            pl.BlockSpec((1, 1), lambda i: (0, 0)),
        ],
        out_specs=pl.BlockSpec((1, 1), lambda i: (0, 0)),
        out_shape=jax.ShapeDtypeStruct((1, 1), jnp.float32),
        scratch_shapes=[pltpu.VMEM((8, F), jnp.float32)],
    )(h, agg, W2, b2, Wf1, bf1, Wf2, bf2)


def kernel(x, edge_index, W1, b1, W2, b2, Wf1, bf1, Wf2, bf2):
    pad_n = E_PAD - N_EDGES
    src = jnp.concatenate(
        [edge_index[0].astype(jnp.int32),
         jnp.zeros((pad_n,), jnp.int32)]).reshape(NW, NBLK, CPB, CH)
    dst = jnp.concatenate(
        [edge_index[1].astype(jnp.int32),
         N_NODES + (jnp.arange(pad_n, dtype=jnp.int32) % N_DUMMY)]
    ).reshape(NW, NBLK, CPB, CH)
    zeros = jnp.zeros((N_NODES, F), jnp.float32)

    agg1 = _seg_sum(x, zeros, src, dst)
    h1 = _lin_relu(x, agg1, W1, b1.reshape(1, F))
    agg2 = _seg_sum(h1, zeros, src, dst)
    return _final(h1, agg2, W2, b2.reshape(1, F), Wf1, bf1.reshape(1, F),
                  Wf2.reshape(1, F), bf2.reshape(1, 1))


# speedup vs baseline: 3.6012x; 3.6012x over previous
import functools

import jax
import jax.numpy as jnp
from jax import lax
from jax.experimental import pallas as pl
from jax.experimental.pallas import tpu as pltpu
from jax.experimental.pallas import tpu_sc as plsc

N_NODES = 10000
N_EDGES = 320000
F = 128

NC = 2
NS = 16
NW = NC * NS
CH = 128
CPB = 8
NBLK = 10
EPWP = NBLK * CPB * CH
E_PAD = NW * EPWP
N_DUMMY = 512
N_ACC = N_NODES + N_DUMMY
CPS = 624
TAIL = N_NODES - CPS * NS
TAIL_OFF = CPS * NS

_mesh = plsc.VectorSubcoreMesh(core_axis_name="c", subcore_axis_name="s")


@functools.partial(
    pl.kernel,
    out_type=jax.ShapeDtypeStruct((NC, N_NODES, F), jnp.float32),
    mesh=_mesh,
    scratch_types=[
        pltpu.VMEM_SHARED((N_ACC, F), jnp.float32),
        pltpu.VMEM((CPB, CH), jnp.int32),
        pltpu.VMEM((CPB, CH), jnp.int32),
        pltpu.VMEM((CPB, CH), jnp.int32),
        pltpu.VMEM((CPB, CH), jnp.int32),
        pltpu.VMEM((CH, F), jnp.float32),
        pltpu.VMEM((CH, F), jnp.float32),
        pltpu.SemaphoreType.DMA,
        pltpu.SemaphoreType.DMA,
        pltpu.SemaphoreType.DMA,
        pltpu.SemaphoreType.DMA,
    ],
)
def _seg_sum(table, zeros, src4, dst4, out, acc, sA, dA, sB, dB, rows0, rows1,
             semiA, semiB, semg0, semg1):
    c = lax.axis_index("c")
    s = lax.axis_index("s")
    w = s * NC + c

    off = pl.multiple_of(s * CPS, 8)
    pltpu.sync_copy(zeros.at[pl.ds(off, CPS)], acc.at[pl.ds(off, CPS)])

    @pl.when(s == NS - 1)
    def _():
        pltpu.sync_copy(zeros.at[pl.ds(TAIL_OFF, TAIL)],
                        acc.at[pl.ds(TAIL_OFF, TAIL)])

    plsc.subcore_barrier()

    rows = (rows0, rows1)
    semg = (semg0, semg1)

    pltpu.async_copy(src4.at[w, 0], sA, semiA)
    pltpu.async_copy(dst4.at[w, 0], dA, semiA)
    pltpu.make_async_copy(src4.at[w, 0], sA, semiA).wait()
    pltpu.make_async_copy(dst4.at[w, 0], dA, semiA).wait()
    pltpu.async_copy(table.at[sA.at[0]], rows0, semg0)

    def pair(t, carry):
        be = 2 * t

        pltpu.async_copy(src4.at[w, be + 1], sB, semiB)
        pltpu.async_copy(dst4.at[w, be + 1], dB, semiB)
        for r in range(CPB):
            cur, nxt = rows[r % 2], rows[(r + 1) % 2]
            if r < CPB - 1:
                pltpu.async_copy(table.at[sA.at[r + 1]], nxt, semg[(r + 1) % 2])
            else:
                pltpu.make_async_copy(src4.at[w, be + 1], sB, semiB).wait()
                pltpu.make_async_copy(dst4.at[w, be + 1], dB, semiB).wait()
                pltpu.async_copy(table.at[sB.at[0]], nxt, semg[(r + 1) % 2])
            pltpu.make_async_copy(table.at[sA.at[r]], cur, semg[r % 2]).wait()
            pltpu.sync_copy(cur, acc.at[dA.at[r]], add=True)

        @pl.when(t < NBLK // 2 - 1)
        def _():
            pltpu.async_copy(src4.at[w, be + 2], sA, semiA)
            pltpu.async_copy(dst4.at[w, be + 2], dA, semiA)

        for r in range(CPB):
            cur, nxt = rows[r % 2], rows[(r + 1) % 2]
            if r < CPB - 1:
                pltpu.async_copy(table.at[sB.at[r + 1]], nxt, semg[(r + 1) % 2])
            else:
                @pl.when(t < NBLK // 2 - 1)
                def _():
                    pltpu.make_async_copy(src4.at[w, be + 2], sA, semiA).wait()
                    pltpu.make_async_copy(dst4.at[w, be + 2], dA, semiA).wait()
                    pltpu.async_copy(table.at[sA.at[0]], nxt,
                                     semg[(r + 1) % 2])
            pltpu.make_async_copy(table.at[sB.at[r]], cur, semg[r % 2]).wait()
            pltpu.sync_copy(cur, acc.at[dB.at[r]], add=True)
        return carry

    lax.fori_loop(0, NBLK // 2, pair, 0)
    plsc.subcore_barrier()

    pltpu.sync_copy(acc.at[pl.ds(off, CPS)], out.at[c, pl.ds(off, CPS)])

    @pl.when(s == NS - 1)
    def _():
        pltpu.sync_copy(acc.at[pl.ds(TAIL_OFF, TAIL)],
                        out.at[c, pl.ds(TAIL_OFF, TAIL)])


R = 1000


def _lin_relu_body(x_ref, agg_ref, w_ref, b_ref, o_ref):
    a = x_ref[...] + agg_ref[0] + agg_ref[1]
    h = jnp.dot(a, w_ref[...], preferred_element_type=jnp.float32) + b_ref[...]
    o_ref[...] = jnp.maximum(h, 0.0)


def _lin_relu(x, agg, W, b):
    return pl.pallas_call(
        _lin_relu_body,
        grid=(N_NODES // R,),
        in_specs=[
            pl.BlockSpec((R, F), lambda i: (i, 0)),
            pl.BlockSpec((NC, R, F), lambda i: (0, i, 0)),
            pl.BlockSpec((F, F), lambda i: (0, 0)),
            pl.BlockSpec((1, F), lambda i: (0, 0)),
        ],
        out_specs=pl.BlockSpec((R, F), lambda i: (i, 0)),
        out_shape=jax.ShapeDtypeStruct((N_NODES, F), jnp.float32),
    )(x, agg, W, b)


def _final_body(h_ref, agg_ref, w2_ref, b2_ref, wf1_ref, bf1_ref, wf2_ref,
                bf2_ref, o_ref, acc_ref):
    i = pl.program_id(0)
    a = h_ref[...] + agg_ref[0] + agg_ref[1]
    h2 = jnp.dot(a, w2_ref[...], preferred_element_type=jnp.float32) + b2_ref[...]
    h2 = jnp.maximum(h2, 0.0)
    part = jnp.sum(h2, axis=0, keepdims=True)

    @pl.when(i == 0)
    def _():
        acc_ref[0:1] = part

    @pl.when(i > 0)
    def _():
        acc_ref[0:1] = acc_ref[0:1] + part

    @pl.when(i == pl.num_programs(0) - 1)
    def _():
        hg = jnp.dot(acc_ref[0:1], wf1_ref[...],
                     preferred_element_type=jnp.float32) + bf1_ref[...]
        hg = jnp.maximum(hg, 0.0)
        z = jnp.sum(hg * wf2_ref[...], axis=1, keepdims=True) + bf2_ref[...]
        o_ref[...] = 1.0 / (1.0 + jnp.exp(-z))


def _final(h, agg, W2, b2, Wf1, bf1, Wf2, bf2):
    return pl.pallas_call(
        _final_body,
        grid=(N_NODES // R,),
        in_specs=[
            pl.BlockSpec((R, F), lambda i: (i, 0)),
            pl.BlockSpec((NC, R, F), lambda i: (0, i, 0)),
            pl.BlockSpec((F, F), lambda i: (0, 0)),
            pl.BlockSpec((1, F), lambda i: (0, 0)),
            pl.BlockSpec((F, F), lambda i: (0, 0)),
            pl.BlockSpec((1, F), lambda i: (0, 0)),
            pl.BlockSpec((1, F), lambda i: (0, 0)),
            pl.BlockSpec((1, 1), lambda i: (0, 0)),
        ],
        out_specs=pl.BlockSpec((1, 1), lambda i: (0, 0)),
        out_shape=jax.ShapeDtypeStruct((1, 1), jnp.float32),
        scratch_shapes=[pltpu.VMEM((8, F), jnp.float32)],
    )(h, agg, W2, b2, Wf1, bf1, Wf2, bf2)


def kernel(x, edge_index, W1, b1, W2, b2, Wf1, bf1, Wf2, bf2):
    ppw = EPWP - N_EDGES // NW
    pad_iota = jnp.arange(NW * ppw, dtype=jnp.int32).reshape(NW, ppw)
    src = jnp.concatenate(
        [edge_index[0].astype(jnp.int32).reshape(NW, N_EDGES // NW),
         pad_iota % N_NODES], axis=1).reshape(NW, NBLK, CPB, CH)
    dst = jnp.concatenate(
        [edge_index[1].astype(jnp.int32).reshape(NW, N_EDGES // NW),
         N_NODES + (pad_iota % N_DUMMY)], axis=1).reshape(NW, NBLK, CPB, CH)
    zeros = jnp.zeros((N_NODES, F), jnp.float32)

    agg1 = _seg_sum(x, zeros, src, dst)
    h1 = _lin_relu(x, agg1, W1, b1.reshape(1, F))
    agg2 = _seg_sum(h1, zeros, src, dst)
    return _final(h1, agg2, W2, b2.reshape(1, F), Wf1, bf1.reshape(1, F),
                  Wf2.reshape(1, F), bf2.reshape(1, 1))


# speedup vs baseline: 3.6557x; 1.0151x over previous
import functools

import jax
import jax.numpy as jnp
from jax import lax
from jax.experimental import pallas as pl
from jax.experimental.pallas import tpu as pltpu
from jax.experimental.pallas import tpu_sc as plsc

N_NODES = 10000
N_EDGES = 320000
F = 128

NC = 2
NS = 16
NW = NC * NS
CH = 128
CPB = 8
NBLK = 10
EPWP = NBLK * CPB * CH
E_PAD = NW * EPWP
N_DUMMY = 512
N_ACC = N_NODES + N_DUMMY
CPS = 624
TAIL = N_NODES - CPS * NS
TAIL_OFF = CPS * NS

_mesh = plsc.VectorSubcoreMesh(core_axis_name="c", subcore_axis_name="s")


@functools.partial(
    pl.kernel,
    out_type=jax.ShapeDtypeStruct((NC, N_NODES, F), jnp.float32),
    mesh=_mesh,
    scratch_types=[
        pltpu.VMEM_SHARED((N_ACC, F), jnp.float32),
        pltpu.VMEM((CPB, CH), jnp.int32),
        pltpu.VMEM((CPB, CH), jnp.int32),
        pltpu.VMEM((CPB, CH), jnp.int32),
        pltpu.VMEM((CPB, CH), jnp.int32),
        pltpu.VMEM((CH, F), jnp.float32),
        pltpu.VMEM((CH, F), jnp.float32),
        pltpu.SemaphoreType.DMA,
        pltpu.SemaphoreType.DMA,
        pltpu.SemaphoreType.DMA,
        pltpu.SemaphoreType.DMA,
    ],
)
def _seg_sum(table, zeros, src4, dst4, out, acc, sA, dA, sB, dB, rows0, rows1,
             semiA, semiB, semg0, semg1):
    c = lax.axis_index("c")
    s = lax.axis_index("s")
    w = s * NC + c

    off = pl.multiple_of(s * CPS, 8)

    @pl.when(c == 0)
    def _():
        pltpu.sync_copy(table.at[pl.ds(off, CPS)], acc.at[pl.ds(off, CPS)])

        @pl.when(s == NS - 1)
        def _():
            pltpu.sync_copy(table.at[pl.ds(TAIL_OFF, TAIL)],
                            acc.at[pl.ds(TAIL_OFF, TAIL)])

    @pl.when(c == 1)
    def _():
        pltpu.sync_copy(zeros, acc.at[pl.ds(off, CPS)])

        @pl.when(s == NS - 1)
        def _():
            pltpu.sync_copy(zeros.at[pl.ds(0, TAIL)],
                            acc.at[pl.ds(TAIL_OFF, TAIL)])

    plsc.subcore_barrier()

    rows = (rows0, rows1)
    semg = (semg0, semg1)

    pltpu.async_copy(src4.at[w, 0], sA, semiA)
    pltpu.async_copy(dst4.at[w, 0], dA, semiA)
    pltpu.make_async_copy(src4.at[w, 0], sA, semiA).wait()
    pltpu.make_async_copy(dst4.at[w, 0], dA, semiA).wait()
    pltpu.async_copy(table.at[sA.at[0]], rows0, semg0)

    def pair(t, carry):
        be = 2 * t

        pltpu.async_copy(src4.at[w, be + 1], sB, semiB)
        pltpu.async_copy(dst4.at[w, be + 1], dB, semiB)
        for r in range(CPB):
            cur, nxt = rows[r % 2], rows[(r + 1) % 2]
            if r < CPB - 1:
                pltpu.async_copy(table.at[sA.at[r + 1]], nxt, semg[(r + 1) % 2])
            else:
                pltpu.make_async_copy(src4.at[w, be + 1], sB, semiB).wait()
                pltpu.make_async_copy(dst4.at[w, be + 1], dB, semiB).wait()
                pltpu.async_copy(table.at[sB.at[0]], nxt, semg[(r + 1) % 2])
            pltpu.make_async_copy(table.at[sA.at[r]], cur, semg[r % 2]).wait()
            pltpu.sync_copy(cur, acc.at[dA.at[r]], add=True)

        @pl.when(t < NBLK // 2 - 1)
        def _():
            pltpu.async_copy(src4.at[w, be + 2], sA, semiA)
            pltpu.async_copy(dst4.at[w, be + 2], dA, semiA)

        for r in range(CPB):
            cur, nxt = rows[r % 2], rows[(r + 1) % 2]
            if r < CPB - 1:
                pltpu.async_copy(table.at[sB.at[r + 1]], nxt, semg[(r + 1) % 2])
            else:
                @pl.when(t < NBLK // 2 - 1)
                def _():
                    pltpu.make_async_copy(src4.at[w, be + 2], sA, semiA).wait()
                    pltpu.make_async_copy(dst4.at[w, be + 2], dA, semiA).wait()
                    pltpu.async_copy(table.at[sA.at[0]], nxt,
                                     semg[(r + 1) % 2])
            pltpu.make_async_copy(table.at[sB.at[r]], cur, semg[r % 2]).wait()
            pltpu.sync_copy(cur, acc.at[dB.at[r]], add=True)
        return carry

    lax.fori_loop(0, NBLK // 2, pair, 0)
    plsc.subcore_barrier()

    pltpu.sync_copy(acc.at[pl.ds(off, CPS)], out.at[c, pl.ds(off, CPS)])

    @pl.when(s == NS - 1)
    def _():
        pltpu.sync_copy(acc.at[pl.ds(TAIL_OFF, TAIL)],
                        out.at[c, pl.ds(TAIL_OFF, TAIL)])


R = 1000


def _lin_relu_body(agg_ref, w_ref, b_ref, o_ref):
    a = agg_ref[0] + agg_ref[1]
    h = jnp.dot(a, w_ref[...], preferred_element_type=jnp.float32) + b_ref[...]
    o_ref[...] = jnp.maximum(h, 0.0)


def _lin_relu(agg, W, b):
    return pl.pallas_call(
        _lin_relu_body,
        grid=(N_NODES // R,),
        in_specs=[
            pl.BlockSpec((NC, R, F), lambda i: (0, i, 0)),
            pl.BlockSpec((F, F), lambda i: (0, 0)),
            pl.BlockSpec((1, F), lambda i: (0, 0)),
        ],
        out_specs=pl.BlockSpec((R, F), lambda i: (i, 0)),
        out_shape=jax.ShapeDtypeStruct((N_NODES, F), jnp.float32),
    )(agg, W, b)


def _final_body(agg_ref, w2_ref, b2_ref, wf1_ref, bf1_ref, wf2_ref,
                bf2_ref, o_ref, acc_ref):
    i = pl.program_id(0)
    a = agg_ref[0] + agg_ref[1]
    h2 = jnp.dot(a, w2_ref[...], preferred_element_type=jnp.float32) + b2_ref[...]
    h2 = jnp.maximum(h2, 0.0)
    part = jnp.sum(h2, axis=0, keepdims=True)

    @pl.when(i == 0)
    def _():
        acc_ref[0:1] = part

    @pl.when(i > 0)
    def _():
        acc_ref[0:1] = acc_ref[0:1] + part

    @pl.when(i == pl.num_programs(0) - 1)
    def _():
        hg = jnp.dot(acc_ref[0:1], wf1_ref[...],
                     preferred_element_type=jnp.float32) + bf1_ref[...]
        hg = jnp.maximum(hg, 0.0)
        z = jnp.sum(hg * wf2_ref[...], axis=1, keepdims=True) + bf2_ref[...]
        o_ref[...] = 1.0 / (1.0 + jnp.exp(-z))


def _final(agg, W2, b2, Wf1, bf1, Wf2, bf2):
    return pl.pallas_call(
        _final_body,
        grid=(N_NODES // R,),
        in_specs=[
            pl.BlockSpec((NC, R, F), lambda i: (0, i, 0)),
            pl.BlockSpec((F, F), lambda i: (0, 0)),
            pl.BlockSpec((1, F), lambda i: (0, 0)),
            pl.BlockSpec((F, F), lambda i: (0, 0)),
            pl.BlockSpec((1, F), lambda i: (0, 0)),
            pl.BlockSpec((1, F), lambda i: (0, 0)),
            pl.BlockSpec((1, 1), lambda i: (0, 0)),
        ],
        out_specs=pl.BlockSpec((1, 1), lambda i: (0, 0)),
        out_shape=jax.ShapeDtypeStruct((1, 1), jnp.float32),
        scratch_shapes=[pltpu.VMEM((8, F), jnp.float32)],
    )(agg, W2, b2, Wf1, bf1, Wf2, bf2)


def kernel(x, edge_index, W1, b1, W2, b2, Wf1, bf1, Wf2, bf2):
    ppw = EPWP - N_EDGES // NW
    pad_iota = jnp.arange(NW * ppw, dtype=jnp.int32).reshape(NW, ppw)
    src = jnp.concatenate(
        [edge_index[0].astype(jnp.int32).reshape(NW, N_EDGES // NW),
         pad_iota % N_NODES], axis=1).reshape(NW, NBLK, CPB, CH)
    dst = jnp.concatenate(
        [edge_index[1].astype(jnp.int32).reshape(NW, N_EDGES // NW),
         N_NODES + (pad_iota % N_DUMMY)], axis=1).reshape(NW, NBLK, CPB, CH)
    zeros = jnp.zeros((CPS, F), jnp.float32)

    agg1 = _seg_sum(x, zeros, src, dst)
    h1 = _lin_relu(agg1, W1, b1.reshape(1, F))
    agg2 = _seg_sum(h1, zeros, src, dst)
    return _final(agg2, W2, b2.reshape(1, F), Wf1, bf1.reshape(1, F),
                  Wf2.reshape(1, F), bf2.reshape(1, 1))


# speedup vs baseline: 3.6907x; 1.0096x over previous
import functools

import jax
import jax.numpy as jnp
from jax import lax
from jax.experimental import pallas as pl
from jax.experimental.pallas import tpu as pltpu
from jax.experimental.pallas import tpu_sc as plsc

N_NODES = 10000
N_EDGES = 320000
F = 128

NC = 2
NS = 16
NW = NC * NS
CH = 128
CPB = 8
NBLK = 10
EPWP = NBLK * CPB * CH
E_PAD = NW * EPWP
N_DUMMY = 512
N_ACC = N_NODES + N_DUMMY
CPS = 624
TAIL = N_NODES - CPS * NS
TAIL_OFF = CPS * NS

_mesh = plsc.VectorSubcoreMesh(core_axis_name="c", subcore_axis_name="s")


@functools.partial(
    pl.kernel,
    out_type=jax.ShapeDtypeStruct((NC, N_NODES, F), jnp.float32),
    mesh=_mesh,
    scratch_types=[
        pltpu.VMEM_SHARED((N_ACC, F), jnp.float32),
        pltpu.VMEM((CPB, CH), jnp.int32),
        pltpu.VMEM((CPB, CH), jnp.int32),
        pltpu.VMEM((CPB, CH), jnp.int32),
        pltpu.VMEM((CPB, CH), jnp.int32),
        pltpu.VMEM((CH, F), jnp.float32),
        pltpu.VMEM((CH, F), jnp.float32),
        pltpu.SemaphoreType.DMA,
        pltpu.SemaphoreType.DMA,
        pltpu.SemaphoreType.DMA,
        pltpu.SemaphoreType.DMA,
    ],
)
def _seg_sum(table, zeros, src4, dst4, out, acc, sA, dA, sB, dB, rows0, rows1,
             semiA, semiB, semg0, semg1):
    c = lax.axis_index("c")
    s = lax.axis_index("s")
    w = s * NC + c

    off = pl.multiple_of(s * CPS, 8)

    @pl.when(c == 0)
    def _():
        pltpu.sync_copy(table.at[pl.ds(off, CPS)], acc.at[pl.ds(off, CPS)])

        @pl.when(s == NS - 1)
        def _():
            pltpu.sync_copy(table.at[pl.ds(TAIL_OFF, TAIL)],
                            acc.at[pl.ds(TAIL_OFF, TAIL)])

    @pl.when(c == 1)
    def _():
        pltpu.sync_copy(zeros, acc.at[pl.ds(off, CPS)])

        @pl.when(s == NS - 1)
        def _():
            pltpu.sync_copy(zeros.at[pl.ds(0, TAIL)],
                            acc.at[pl.ds(TAIL_OFF, TAIL)])

    plsc.subcore_barrier()

    rows = (rows0, rows1)
    semg = (semg0, semg1)

    pltpu.async_copy(src4.at[w, 0], sA, semiA)
    pltpu.async_copy(dst4.at[w, 0], dA, semiA)
    pltpu.make_async_copy(src4.at[w, 0], sA, semiA).wait()
    pltpu.make_async_copy(dst4.at[w, 0], dA, semiA).wait()
    pltpu.async_copy(table.at[sA.at[0]], rows0, semg0)

    def pair(t, carry):
        be = 2 * t

        pltpu.async_copy(src4.at[w, be + 1], sB, semiB)
        pltpu.async_copy(dst4.at[w, be + 1], dB, semiB)
        for r in range(CPB):
            cur, nxt = rows[r % 2], rows[(r + 1) % 2]
            if r < CPB - 1:
                pltpu.async_copy(table.at[sA.at[r + 1]], nxt, semg[(r + 1) % 2])
            else:
                pltpu.make_async_copy(src4.at[w, be + 1], sB, semiB).wait()
                pltpu.make_async_copy(dst4.at[w, be + 1], dB, semiB).wait()
                pltpu.async_copy(table.at[sB.at[0]], nxt, semg[(r + 1) % 2])
            pltpu.make_async_copy(table.at[sA.at[r]], cur, semg[r % 2]).wait()
            pltpu.sync_copy(cur, acc.at[dA.at[r]], add=True)

        @pl.when(t < NBLK // 2 - 1)
        def _():
            pltpu.async_copy(src4.at[w, be + 2], sA, semiA)
            pltpu.async_copy(dst4.at[w, be + 2], dA, semiA)

        for r in range(CPB):
            cur, nxt = rows[r % 2], rows[(r + 1) % 2]
            if r < CPB - 1:
                pltpu.async_copy(table.at[sB.at[r + 1]], nxt, semg[(r + 1) % 2])
            else:
                @pl.when(t < NBLK // 2 - 1)
                def _():
                    pltpu.make_async_copy(src4.at[w, be + 2], sA, semiA).wait()
                    pltpu.make_async_copy(dst4.at[w, be + 2], dA, semiA).wait()
                    pltpu.async_copy(table.at[sA.at[0]], nxt,
                                     semg[(r + 1) % 2])
            pltpu.make_async_copy(table.at[sB.at[r]], cur, semg[r % 2]).wait()
            pltpu.sync_copy(cur, acc.at[dB.at[r]], add=True)
        return carry

    lax.fori_loop(0, NBLK // 2, pair, 0)
    plsc.subcore_barrier()

    pltpu.sync_copy(acc.at[pl.ds(off, CPS)], out.at[c, pl.ds(off, CPS)])

    @pl.when(s == NS - 1)
    def _():
        pltpu.sync_copy(acc.at[pl.ds(TAIL_OFF, TAIL)],
                        out.at[c, pl.ds(TAIL_OFF, TAIL)])


R = 2000


def _lin_relu_body(agg_ref, w_ref, b_ref, o_ref):
    a = agg_ref[0] + agg_ref[1]
    h = jnp.dot(a, w_ref[...], preferred_element_type=jnp.float32) + b_ref[...]
    o_ref[...] = jnp.maximum(h, 0.0)


def _lin_relu(agg, W, b):
    return pl.pallas_call(
        _lin_relu_body,
        grid=(N_NODES // R,),
        in_specs=[
            pl.BlockSpec((NC, R, F), lambda i: (0, i, 0)),
            pl.BlockSpec((F, F), lambda i: (0, 0)),
            pl.BlockSpec((1, F), lambda i: (0, 0)),
        ],
        out_specs=pl.BlockSpec((R, F), lambda i: (i, 0)),
        out_shape=jax.ShapeDtypeStruct((N_NODES, F), jnp.float32),
    )(agg, W, b)


def _final_body(agg_ref, w2_ref, b2_ref, wf1_ref, bf1_ref, wf2_ref,
                bf2_ref, o_ref, acc_ref):
    i = pl.program_id(0)
    a = agg_ref[0] + agg_ref[1]
    h2 = jnp.dot(a, w2_ref[...], preferred_element_type=jnp.float32) + b2_ref[...]
    h2 = jnp.maximum(h2, 0.0)
    part = jnp.sum(h2, axis=0, keepdims=True)

    @pl.when(i == 0)
    def _():
        acc_ref[0:1] = part

    @pl.when(i > 0)
    def _():
        acc_ref[0:1] = acc_ref[0:1] + part

    @pl.when(i == pl.num_programs(0) - 1)
    def _():
        hg = jnp.dot(acc_ref[0:1], wf1_ref[...],
                     preferred_element_type=jnp.float32) + bf1_ref[...]
        hg = jnp.maximum(hg, 0.0)
        z = jnp.sum(hg * wf2_ref[...], axis=1, keepdims=True) + bf2_ref[...]
        o_ref[...] = 1.0 / (1.0 + jnp.exp(-z))


def _final(agg, W2, b2, Wf1, bf1, Wf2, bf2):
    return pl.pallas_call(
        _final_body,
        grid=(N_NODES // R,),
        in_specs=[
            pl.BlockSpec((NC, R, F), lambda i: (0, i, 0)),
            pl.BlockSpec((F, F), lambda i: (0, 0)),
            pl.BlockSpec((1, F), lambda i: (0, 0)),
            pl.BlockSpec((F, F), lambda i: (0, 0)),
            pl.BlockSpec((1, F), lambda i: (0, 0)),
            pl.BlockSpec((1, F), lambda i: (0, 0)),
            pl.BlockSpec((1, 1), lambda i: (0, 0)),
        ],
        out_specs=pl.BlockSpec((1, 1), lambda i: (0, 0)),
        out_shape=jax.ShapeDtypeStruct((1, 1), jnp.float32),
        scratch_shapes=[pltpu.VMEM((8, F), jnp.float32)],
    )(agg, W2, b2, Wf1, bf1, Wf2, bf2)


def kernel(x, edge_index, W1, b1, W2, b2, Wf1, bf1, Wf2, bf2):
    ppw = EPWP - N_EDGES // NW
    pad_iota = jnp.arange(NW * ppw, dtype=jnp.int32).reshape(NW, ppw)
    src = jnp.concatenate(
        [edge_index[0].astype(jnp.int32).reshape(NW, N_EDGES // NW),
         pad_iota % N_NODES], axis=1).reshape(NW, NBLK, CPB, CH)
    dst = jnp.concatenate(
        [edge_index[1].astype(jnp.int32).reshape(NW, N_EDGES // NW),
         N_NODES + (pad_iota % N_DUMMY)], axis=1).reshape(NW, NBLK, CPB, CH)
    zeros = jnp.zeros((CPS, F), jnp.float32)

    agg1 = _seg_sum(x, zeros, src, dst)
    h1 = _lin_relu(agg1, W1, b1.reshape(1, F))
    agg2 = _seg_sum(h1, zeros, src, dst)
    return _final(agg2, W2, b2.reshape(1, F), Wf1, bf1.reshape(1, F),
                  Wf2.reshape(1, F), bf2.reshape(1, 1))


# speedup vs baseline: 3.7029x; 1.0033x over previous
import functools

import jax
import jax.numpy as jnp
from jax import lax
from jax.experimental import pallas as pl
from jax.experimental.pallas import tpu as pltpu
from jax.experimental.pallas import tpu_sc as plsc

N_NODES = 10000
N_EDGES = 320000
F = 128

NC = 2
NS = 16
NW = NC * NS
CH = 128
CPB = 8
NBLK = 10
EPWP = NBLK * CPB * CH
E_PAD = NW * EPWP
N_DUMMY = 512
N_ACC = N_NODES + N_DUMMY
CPS = 624
TAIL = N_NODES - CPS * NS
TAIL_OFF = CPS * NS

_mesh = plsc.VectorSubcoreMesh(core_axis_name="c", subcore_axis_name="s")


@functools.partial(
    pl.kernel,
    out_type=jax.ShapeDtypeStruct((NC, N_NODES, F), jnp.float32),
    mesh=_mesh,
    scratch_types=[
        pltpu.VMEM_SHARED((N_ACC, F), jnp.float32),
        pltpu.VMEM((CPB, CH), jnp.int32),
        pltpu.VMEM((CPB, CH), jnp.int32),
        pltpu.VMEM((CPB, CH), jnp.int32),
        pltpu.VMEM((CPB, CH), jnp.int32),
        pltpu.VMEM((CH, F), jnp.float32),
        pltpu.VMEM((CH, F), jnp.float32),
        pltpu.SemaphoreType.DMA,
        pltpu.SemaphoreType.DMA,
        pltpu.SemaphoreType.DMA,
        pltpu.SemaphoreType.DMA,
    ],
)
def _seg_sum(table, zeros, src4, dst4, out, acc, sA, dA, sB, dB, rows0, rows1,
             semiA, semiB, semg0, semg1):
    c = lax.axis_index("c")
    s = lax.axis_index("s")
    w = s * NC + c

    off = pl.multiple_of(s * CPS, 8)

    @pl.when(c == 0)
    def _():
        pltpu.sync_copy(table.at[pl.ds(off, CPS)], acc.at[pl.ds(off, CPS)])

        @pl.when(s == NS - 1)
        def _():
            pltpu.sync_copy(table.at[pl.ds(TAIL_OFF, TAIL)],
                            acc.at[pl.ds(TAIL_OFF, TAIL)])

    @pl.when(c == 1)
    def _():
        pltpu.sync_copy(zeros, acc.at[pl.ds(off, CPS)])

        @pl.when(s == NS - 1)
        def _():
            pltpu.sync_copy(zeros.at[pl.ds(0, TAIL)],
                            acc.at[pl.ds(TAIL_OFF, TAIL)])

    plsc.subcore_barrier()

    rows = (rows0, rows1)
    semg = (semg0, semg1)
    HC = CH // 2

    def gissue(sref, r, buf, sem):
        pltpu.async_copy(table.at[sref.at[r, pl.ds(0, HC)]],
                         buf.at[pl.ds(0, HC)], sem)
        pltpu.async_copy(table.at[sref.at[r, pl.ds(HC, HC)]],
                         buf.at[pl.ds(HC, HC)], sem)

    def gwait(sref, r, buf, sem):
        pltpu.make_async_copy(table.at[sref.at[r, pl.ds(0, HC)]],
                              buf.at[pl.ds(0, HC)], sem).wait()
        pltpu.make_async_copy(table.at[sref.at[r, pl.ds(HC, HC)]],
                              buf.at[pl.ds(HC, HC)], sem).wait()

    pltpu.async_copy(src4.at[w, 0], sA, semiA)
    pltpu.async_copy(dst4.at[w, 0], dA, semiA)
    pltpu.make_async_copy(src4.at[w, 0], sA, semiA).wait()
    pltpu.make_async_copy(dst4.at[w, 0], dA, semiA).wait()
    gissue(sA, 0, rows0, semg0)

    def pair(t, carry):
        be = 2 * t

        pltpu.async_copy(src4.at[w, be + 1], sB, semiB)
        pltpu.async_copy(dst4.at[w, be + 1], dB, semiB)
        for r in range(CPB):
            cur, nxt = rows[r % 2], rows[(r + 1) % 2]
            if r < CPB - 1:
                gissue(sA, r + 1, nxt, semg[(r + 1) % 2])
            else:
                pltpu.make_async_copy(src4.at[w, be + 1], sB, semiB).wait()
                pltpu.make_async_copy(dst4.at[w, be + 1], dB, semiB).wait()
                gissue(sB, 0, nxt, semg[(r + 1) % 2])
            gwait(sA, r, cur, semg[r % 2])
            pltpu.sync_copy(cur, acc.at[dA.at[r]], add=True)

        @pl.when(t < NBLK // 2 - 1)
        def _():
            pltpu.async_copy(src4.at[w, be + 2], sA, semiA)
            pltpu.async_copy(dst4.at[w, be + 2], dA, semiA)

        for r in range(CPB):
            cur, nxt = rows[r % 2], rows[(r + 1) % 2]
            if r < CPB - 1:
                gissue(sB, r + 1, nxt, semg[(r + 1) % 2])
            else:
                @pl.when(t < NBLK // 2 - 1)
                def _():
                    pltpu.make_async_copy(src4.at[w, be + 2], sA, semiA).wait()
                    pltpu.make_async_copy(dst4.at[w, be + 2], dA, semiA).wait()
                    gissue(sA, 0, nxt, semg[(r + 1) % 2])
            gwait(sB, r, cur, semg[r % 2])
            pltpu.sync_copy(cur, acc.at[dB.at[r]], add=True)
        return carry

    lax.fori_loop(0, NBLK // 2, pair, 0)
    plsc.subcore_barrier()

    pltpu.sync_copy(acc.at[pl.ds(off, CPS)], out.at[c, pl.ds(off, CPS)])

    @pl.when(s == NS - 1)
    def _():
        pltpu.sync_copy(acc.at[pl.ds(TAIL_OFF, TAIL)],
                        out.at[c, pl.ds(TAIL_OFF, TAIL)])


R = 2000


def _lin_relu_body(agg_ref, w_ref, b_ref, o_ref):
    a = agg_ref[0] + agg_ref[1]
    h = jnp.dot(a, w_ref[...], preferred_element_type=jnp.float32) + b_ref[...]
    o_ref[...] = jnp.maximum(h, 0.0)


def _lin_relu(agg, W, b):
    return pl.pallas_call(
        _lin_relu_body,
        grid=(N_NODES // R,),
        in_specs=[
            pl.BlockSpec((NC, R, F), lambda i: (0, i, 0)),
            pl.BlockSpec((F, F), lambda i: (0, 0)),
            pl.BlockSpec((1, F), lambda i: (0, 0)),
        ],
        out_specs=pl.BlockSpec((R, F), lambda i: (i, 0)),
        out_shape=jax.ShapeDtypeStruct((N_NODES, F), jnp.float32),
    )(agg, W, b)


def _final_body(agg_ref, w2_ref, b2_ref, wf1_ref, bf1_ref, wf2_ref,
                bf2_ref, o_ref, acc_ref):
    i = pl.program_id(0)
    a = agg_ref[0] + agg_ref[1]
    h2 = jnp.dot(a, w2_ref[...], preferred_element_type=jnp.float32) + b2_ref[...]
    h2 = jnp.maximum(h2, 0.0)
    part = jnp.sum(h2, axis=0, keepdims=True)

    @pl.when(i == 0)
    def _():
        acc_ref[0:1] = part

    @pl.when(i > 0)
    def _():
        acc_ref[0:1] = acc_ref[0:1] + part

    @pl.when(i == pl.num_programs(0) - 1)
    def _():
        hg = jnp.dot(acc_ref[0:1], wf1_ref[...],
                     preferred_element_type=jnp.float32) + bf1_ref[...]
        hg = jnp.maximum(hg, 0.0)
        z = jnp.sum(hg * wf2_ref[...], axis=1, keepdims=True) + bf2_ref[...]
        o_ref[...] = 1.0 / (1.0 + jnp.exp(-z))


def _final(agg, W2, b2, Wf1, bf1, Wf2, bf2):
    return pl.pallas_call(
        _final_body,
        grid=(N_NODES // R,),
        in_specs=[
            pl.BlockSpec((NC, R, F), lambda i: (0, i, 0)),
            pl.BlockSpec((F, F), lambda i: (0, 0)),
            pl.BlockSpec((1, F), lambda i: (0, 0)),
            pl.BlockSpec((F, F), lambda i: (0, 0)),
            pl.BlockSpec((1, F), lambda i: (0, 0)),
            pl.BlockSpec((1, F), lambda i: (0, 0)),
            pl.BlockSpec((1, 1), lambda i: (0, 0)),
        ],
        out_specs=pl.BlockSpec((1, 1), lambda i: (0, 0)),
        out_shape=jax.ShapeDtypeStruct((1, 1), jnp.float32),
        scratch_shapes=[pltpu.VMEM((8, F), jnp.float32)],
    )(agg, W2, b2, Wf1, bf1, Wf2, bf2)


def kernel(x, edge_index, W1, b1, W2, b2, Wf1, bf1, Wf2, bf2):
    ppw = EPWP - N_EDGES // NW
    pad_iota = jnp.arange(NW * ppw, dtype=jnp.int32).reshape(NW, ppw)
    src = jnp.concatenate(
        [edge_index[0].astype(jnp.int32).reshape(NW, N_EDGES // NW),
         pad_iota % N_NODES], axis=1).reshape(NW, NBLK, CPB, CH)
    dst = jnp.concatenate(
        [edge_index[1].astype(jnp.int32).reshape(NW, N_EDGES // NW),
         N_NODES + (pad_iota % N_DUMMY)], axis=1).reshape(NW, NBLK, CPB, CH)
    zeros = jnp.zeros((CPS, F), jnp.float32)

    agg1 = _seg_sum(x, zeros, src, dst)
    h1 = _lin_relu(agg1, W1, b1.reshape(1, F))
    agg2 = _seg_sum(h1, zeros, src, dst)
    return _final(agg2, W2, b2.reshape(1, F), Wf1, bf1.reshape(1, F),
                  Wf2.reshape(1, F), bf2.reshape(1, 1))
